# baseline (speedup 1.0000x reference)
import functools

import jax
import jax.numpy as jnp
from jax import lax
from jax.experimental import pallas as pl
from jax.experimental.pallas import tpu as pltpu
from jax.experimental.pallas import tpu_sc as plsc

N_NODES = 10000
N_EDGES = 320000
D = 128
N_GRAPHS = 64

NC = 2
NS = 16
NW = NC * NS
EPW = N_EDGES // NW
WIN = 128
NWIN = 80
NWX = NWIN + 2
EPW_PAD = NWIN * WIN
PAD = EPW_PAD - EPW
NSC = N_NODES + 16
NROW = 10240
RPS = NROW // NS


def _tc_first_body(x_ref, w_ref, as_ref, ad_ref, h_ref, asrc_ref, adst_ref,
                   gub_ref):
    h = jnp.dot(x_ref[...], w_ref[...], preferred_element_type=jnp.float32)
    h_ref[...] = h
    asrc = jnp.dot(h, as_ref[...], preferred_element_type=jnp.float32)
    adst = jnp.dot(h, ad_ref[...], preferred_element_type=jnp.float32)
    pad = jnp.zeros((NSC - N_NODES,), jnp.float32)
    asrc_ref[...] = jnp.concatenate([asrc, pad])
    adst_ref[...] = jnp.concatenate([adst, pad])
    ub = jnp.max(asrc) + jnp.max(adst)
    gub = jnp.where(ub >= 0, ub, 0.2 * ub)
    gub_ref[...] = jnp.full((128,), gub, jnp.float32)


def _tc_mid_body(acc_ref, den_ref, b_ref, w_ref, as_ref, ad_ref,
                 h_ref, asrc_ref, adst_ref, gub_ref):
    accs = acc_ref[0, :N_NODES, :] + acc_ref[1, :N_NODES, :]
    dens = den_ref[0, :N_NODES] + den_ref[1, :N_NODES]
    prev = accs / (dens + 1e-16)[:, None] + b_ref[...][None, :]
    prev = jnp.maximum(prev, 0.0)
    h = jnp.dot(prev, w_ref[...], preferred_element_type=jnp.float32)
    h_ref[...] = h
    asrc = jnp.dot(h, as_ref[...], preferred_element_type=jnp.float32)
    adst = jnp.dot(h, ad_ref[...], preferred_element_type=jnp.float32)
    pad = jnp.zeros((NSC - N_NODES,), jnp.float32)
    asrc_ref[...] = jnp.concatenate([asrc, pad])
    adst_ref[...] = jnp.concatenate([adst, pad])
    ub = jnp.max(asrc) + jnp.max(adst)
    gub = jnp.where(ub >= 0, ub, 0.2 * ub)
    gub_ref[...] = jnp.full((128,), gub, jnp.float32)


def _tc_final_body(acc_ref, den_ref, b_ref, batch_ref, lw_ref, lb_ref,
                   out_ref):
    accs = acc_ref[0, :N_NODES, :] + acc_ref[1, :N_NODES, :]
    dens = den_ref[0, :N_NODES] + den_ref[1, :N_NODES]
    node = accs / (dens + 1e-16)[:, None]
    gids = lax.broadcasted_iota(jnp.int32, (N_NODES, N_GRAPHS), 1)
    onehot = (batch_ref[...][:, None] == gids).astype(jnp.float32)
    pooled = lax.dot_general(onehot, node, (((0,), (0,)), ((), ())),
                             preferred_element_type=jnp.float32)
    cnt = jnp.sum(onehot, axis=0)
    pooled = pooled / jnp.maximum(cnt, 1.0)[:, None] + b_ref[...][None, :]
    out_ref[...] = (jnp.dot(pooled, lw_ref[...],
                            preferred_element_type=jnp.float32)
                    + lb_ref[...][None, :])


_TC_PARAMS = pltpu.CompilerParams(vmem_limit_bytes=100 * 1024 * 1024)


def _tc_first(x, w, a_s, a_d):
    return pl.pallas_call(
        _tc_first_body,
        out_shape=(
            jax.ShapeDtypeStruct((N_NODES, D), jnp.float32),
            jax.ShapeDtypeStruct((NSC,), jnp.float32),
            jax.ShapeDtypeStruct((NSC,), jnp.float32),
            jax.ShapeDtypeStruct((128,), jnp.float32),
        ),
        compiler_params=_TC_PARAMS,
    )(x, w, a_s, a_d)


def _tc_mid(acc, den, b, w, a_s, a_d):
    return pl.pallas_call(
        _tc_mid_body,
        out_shape=(
            jax.ShapeDtypeStruct((N_NODES, D), jnp.float32),
            jax.ShapeDtypeStruct((NSC,), jnp.float32),
            jax.ShapeDtypeStruct((NSC,), jnp.float32),
            jax.ShapeDtypeStruct((128,), jnp.float32),
        ),
        compiler_params=_TC_PARAMS,
    )(acc, den, b, w, a_s, a_d)


def _tc_final(acc, den, b, batch_i32, lin_w, lin_b):
    return pl.pallas_call(
        _tc_final_body,
        out_shape=jax.ShapeDtypeStruct((N_GRAPHS, D), jnp.float32),
        compiler_params=_TC_PARAMS,
    )(acc, den, b, batch_i32, lin_w, lin_b)


def _sc_body(h_hbm, asrc_hbm, adst_hbm, gub_hbm, eidx_hbm,
             zacc_hbm, zden_hbm, acc_out, den_out,
             ibuf, rbuf, asv, adv, gub_v, wwin, acc_sh, den_sh,
             isem, gsem):
    c = lax.axis_index("c")
    s = lax.axis_index("s")
    w = c * NS + s

    def idx_copy(j, b):
        return pltpu.make_async_copy(eidx_hbm.at[w].at[j], ibuf.at[b],
                                     isem.at[b])

    def g_copies(j, b):
        si = ibuf.at[b].at[0]
        di = ibuf.at[b].at[1]
        return (pltpu.make_async_copy(h_hbm.at[si], rbuf.at[b], gsem.at[b]),
                pltpu.make_async_copy(asrc_hbm.at[si], asv.at[b],
                                      gsem.at[b]),
                pltpu.make_async_copy(adst_hbm.at[di], adv.at[b],
                                      gsem.at[b]))

    pltpu.sync_copy(gub_hbm.at[pl.ds(0, 16)], gub_v)
    pltpu.sync_copy(zacc_hbm.at[pl.ds(s * RPS, RPS)],
                    acc_sh.at[pl.ds(s * RPS, RPS)])
    pltpu.sync_copy(zden_hbm.at[pl.ds(s * RPS, RPS)],
                    den_sh.at[pl.ds(s * RPS, RPS)])
    plsc.subcore_barrier()
    gvec = gub_v[...]

    idx_copy(0, 0).start()
    idx_copy(0, 0).wait()
    for d in g_copies(0, 0):
        d.start()
    idx_copy(1, 1).start()

    def win_body(j, carry):
        b = lax.rem(j, 2)
        nb = 1 - b
        idx_copy(j + 1, nb).wait()
        for d in g_copies(j + 1, nb):
            d.start()
        for d in g_copies(j, b):
            d.wait()
        for g in range(WIN // 16):
            sl = pl.ds(g * 16, 16)
            al = asv[b, sl] + adv[b, sl]
            al = jnp.where(al >= 0, al, 0.2 * al)
            wwin[sl] = jnp.exp(al - gvec)
        def e_body(e, carry2):
            we = plsc.load_gather(wwin, [jnp.full((16,), e, jnp.int32)])
            for g2 in range(D // 16):
                sl2 = pl.ds(g2 * 16, 16)
                rbuf[b, e, sl2] = rbuf[b, e, sl2] * we
            return carry2
        lax.fori_loop(0, WIN, e_body, 0, unroll=False)
        pltpu.sync_copy(rbuf.at[b], acc_sh.at[ibuf.at[b].at[1]], add=True)
        pltpu.sync_copy(wwin, den_sh.at[ibuf.at[b].at[1]], add=True)
        idx_copy(j + 2, b).start()
        return carry

    lax.fori_loop(0, NWIN, win_body, 0, unroll=False)
    for d in g_copies(NWIN, lax.rem(NWIN, 2)):
        d.wait()
    idx_copy(NWIN + 1, lax.rem(NWIN + 1, 2)).wait()
    plsc.subcore_barrier()
    pltpu.sync_copy(acc_sh.at[pl.ds(s * RPS, RPS)],
                    acc_out.at[c].at[pl.ds(s * RPS, RPS)])
    pltpu.sync_copy(den_sh.at[pl.ds(s * RPS, RPS)],
                    den_out.at[c].at[pl.ds(s * RPS, RPS)])


_sc_layer = pl.kernel(
    _sc_body,
    out_type=(
        jax.ShapeDtypeStruct((NC, NROW, D), jnp.float32),
        jax.ShapeDtypeStruct((NC, NROW), jnp.float32),
    ),
    mesh=plsc.VectorSubcoreMesh(core_axis_name="c", subcore_axis_name="s",
                                num_cores=NC, num_subcores=NS),
    compiler_params=pltpu.CompilerParams(needs_layout_passes=False),
    scratch_types=[
        pltpu.VMEM((2, 2, WIN), jnp.int32),
        pltpu.VMEM((2, WIN, D), jnp.float32),
        pltpu.VMEM((2, WIN), jnp.float32),
        pltpu.VMEM((2, WIN), jnp.float32),
        pltpu.VMEM((16,), jnp.float32),
        pltpu.VMEM((WIN,), jnp.float32),
        pltpu.VMEM_SHARED((NROW, D), jnp.float32),
        pltpu.VMEM_SHARED((NROW,), jnp.float32),
        pltpu.SemaphoreType.DMA((2,)),
        pltpu.SemaphoreType.DMA((2,)),
    ],
)


def kernel(x, edge_index, edge_attr, batch,
           W1, b1, as1, ad1, W2, b2, as2, ad2, W3, b3, as3, ad3,
           lin_W, lin_b):
    del edge_attr
    src = edge_index[0].astype(jnp.int32).reshape(NW, EPW)
    dst = edge_index[1].astype(jnp.int32).reshape(NW, EPW)
    npad = NWX * WIN - EPW
    pad_src = jnp.zeros((NW, npad), jnp.int32)
    pad_dst = jnp.broadcast_to(
        N_NODES + (jnp.arange(npad, dtype=jnp.int32) % (NSC - N_NODES)),
        (NW, npad))
    srcw = jnp.concatenate([src, pad_src], axis=1).reshape(NW, NWX, WIN)
    dstw = jnp.concatenate([dst, pad_dst], axis=1).reshape(NW, NWX, WIN)
    eidx = jnp.stack([srcw, dstw], axis=2)
    zacc = jnp.zeros((NROW, D), jnp.float32)
    zden = jnp.zeros((NROW,), jnp.float32)
    batch_i32 = batch.astype(jnp.int32)

    h, asrc, adst, gub = _tc_first(x, W1, as1, ad1)
    acc, den = _sc_layer(h, asrc, adst, gub, eidx, zacc, zden)
    h, asrc, adst, gub = _tc_mid(acc, den, b1, W2, as2, ad2)
    acc, den = _sc_layer(h, asrc, adst, gub, eidx, zacc, zden)
    h, asrc, adst, gub = _tc_mid(acc, den, b2, W3, as3, ad3)
    acc, den = _sc_layer(h, asrc, adst, gub, eidx, zacc, zden)
    return _tc_final(acc, den, b3, batch_i32, lin_W, lin_b)


# speedup vs baseline: 1.0722x; 1.0722x over previous
import functools

import jax
import jax.numpy as jnp
from jax import lax
from jax.experimental import pallas as pl
from jax.experimental.pallas import tpu as pltpu
from jax.experimental.pallas import tpu_sc as plsc

N_NODES = 10000
N_EDGES = 320000
D = 128
N_GRAPHS = 64

NC = 2
NS = 16
NW = NC * NS
EPW = N_EDGES // NW
WIN = 128
NWIN = 80
NWX = NWIN + 3
EPW_PAD = NWIN * WIN
PAD = EPW_PAD - EPW
NSC = N_NODES + 16
NROW = 10240
RPS = NROW // NS


def _tc_first_body(x_ref, w_ref, as_ref, ad_ref, h_ref, asrc_ref, adst_ref,
                   gub_ref):
    h = jnp.dot(x_ref[...], w_ref[...], preferred_element_type=jnp.float32)
    h_ref[...] = h
    asrc = jnp.dot(h, as_ref[...], preferred_element_type=jnp.float32)
    adst = jnp.dot(h, ad_ref[...], preferred_element_type=jnp.float32)
    pad = jnp.zeros((NSC - N_NODES,), jnp.float32)
    asrc_ref[...] = jnp.concatenate([asrc, pad])
    adst_ref[...] = jnp.concatenate([adst, pad])
    ub = jnp.max(asrc) + jnp.max(adst)
    gub = jnp.where(ub >= 0, ub, 0.2 * ub)
    gub_ref[...] = jnp.full((128,), gub, jnp.float32)


def _tc_mid_body(acc_ref, den_ref, b_ref, w_ref, as_ref, ad_ref,
                 h_ref, asrc_ref, adst_ref, gub_ref):
    accs = acc_ref[0, :N_NODES, :] + acc_ref[1, :N_NODES, :]
    dens = den_ref[0, :N_NODES] + den_ref[1, :N_NODES]
    prev = accs / (dens + 1e-16)[:, None] + b_ref[...][None, :]
    prev = jnp.maximum(prev, 0.0)
    h = jnp.dot(prev, w_ref[...], preferred_element_type=jnp.float32)
    h_ref[...] = h
    asrc = jnp.dot(h, as_ref[...], preferred_element_type=jnp.float32)
    adst = jnp.dot(h, ad_ref[...], preferred_element_type=jnp.float32)
    pad = jnp.zeros((NSC - N_NODES,), jnp.float32)
    asrc_ref[...] = jnp.concatenate([asrc, pad])
    adst_ref[...] = jnp.concatenate([adst, pad])
    ub = jnp.max(asrc) + jnp.max(adst)
    gub = jnp.where(ub >= 0, ub, 0.2 * ub)
    gub_ref[...] = jnp.full((128,), gub, jnp.float32)


def _tc_final_body(acc_ref, den_ref, b_ref, batch_ref, lw_ref, lb_ref,
                   out_ref):
    accs = acc_ref[0, :N_NODES, :] + acc_ref[1, :N_NODES, :]
    dens = den_ref[0, :N_NODES] + den_ref[1, :N_NODES]
    node = accs / (dens + 1e-16)[:, None]
    gids = lax.broadcasted_iota(jnp.int32, (N_NODES, N_GRAPHS), 1)
    onehot = (batch_ref[...][:, None] == gids).astype(jnp.float32)
    pooled = lax.dot_general(onehot, node, (((0,), (0,)), ((), ())),
                             preferred_element_type=jnp.float32)
    cnt = jnp.sum(onehot, axis=0)
    pooled = pooled / jnp.maximum(cnt, 1.0)[:, None] + b_ref[...][None, :]
    out_ref[...] = (jnp.dot(pooled, lw_ref[...],
                            preferred_element_type=jnp.float32)
                    + lb_ref[...][None, :])


_TC_PARAMS = pltpu.CompilerParams(vmem_limit_bytes=100 * 1024 * 1024)


def _tc_first(x, w, a_s, a_d):
    return pl.pallas_call(
        _tc_first_body,
        out_shape=(
            jax.ShapeDtypeStruct((N_NODES, D), jnp.float32),
            jax.ShapeDtypeStruct((NSC,), jnp.float32),
            jax.ShapeDtypeStruct((NSC,), jnp.float32),
            jax.ShapeDtypeStruct((128,), jnp.float32),
        ),
        compiler_params=_TC_PARAMS,
    )(x, w, a_s, a_d)


def _tc_mid(acc, den, b, w, a_s, a_d):
    return pl.pallas_call(
        _tc_mid_body,
        out_shape=(
            jax.ShapeDtypeStruct((N_NODES, D), jnp.float32),
            jax.ShapeDtypeStruct((NSC,), jnp.float32),
            jax.ShapeDtypeStruct((NSC,), jnp.float32),
            jax.ShapeDtypeStruct((128,), jnp.float32),
        ),
        compiler_params=_TC_PARAMS,
    )(acc, den, b, w, a_s, a_d)


def _tc_final(acc, den, b, batch_i32, lin_w, lin_b):
    return pl.pallas_call(
        _tc_final_body,
        out_shape=jax.ShapeDtypeStruct((N_GRAPHS, D), jnp.float32),
        compiler_params=_TC_PARAMS,
    )(acc, den, b, batch_i32, lin_w, lin_b)


def _sc_body(h_hbm, asrc_hbm, adst_hbm, gub_hbm, eidx_hbm,
             zacc_hbm, zden_hbm, acc_out, den_out,
             ibuf, rbuf, asv, adv, gub_v, wwin, acc_sh, den_sh,
             isem, gsem, ssem):
    c = lax.axis_index("c")
    s = lax.axis_index("s")
    w = c * NS + s

    def idx_copy(j, b4):
        return pltpu.make_async_copy(eidx_hbm.at[w].at[j], ibuf.at[b4],
                                     isem.at[b4])

    def g_copies(j, b):
        b4 = lax.rem(j, 4)
        si = ibuf.at[b4].at[0]
        di = ibuf.at[b4].at[1]
        return (pltpu.make_async_copy(h_hbm.at[si], rbuf.at[b], gsem.at[b]),
                pltpu.make_async_copy(asrc_hbm.at[si], asv.at[b],
                                      gsem.at[b]),
                pltpu.make_async_copy(adst_hbm.at[di], adv.at[b],
                                      gsem.at[b]))

    def s_copies(j, b):
        di = ibuf.at[lax.rem(j, 4)].at[1]
        wsl = pl.ds(pl.multiple_of(b * WIN, WIN), WIN)
        return (pltpu.make_async_copy(rbuf.at[b], acc_sh.at[di],
                                      ssem.at[b]),
                pltpu.make_async_copy(wwin.at[wsl], den_sh.at[di],
                                      ssem.at[b]))

    pltpu.sync_copy(gub_hbm.at[pl.ds(0, 16)], gub_v)
    pltpu.sync_copy(zacc_hbm.at[pl.ds(s * RPS, RPS)],
                    acc_sh.at[pl.ds(s * RPS, RPS)])
    pltpu.sync_copy(zden_hbm.at[pl.ds(s * RPS, RPS)],
                    den_sh.at[pl.ds(s * RPS, RPS)])
    plsc.subcore_barrier()
    gvec = gub_v[...]

    for jj in range(3):
        idx_copy(jj, jj).start()
    idx_copy(0, 0).wait()
    for d in g_copies(0, 0):
        d.start()

    def win_body(j, carry):
        b = lax.rem(j, 2)
        nb = 1 - b
        @pl.when(j > 0)
        def _():
            for d in s_copies(j - 1, nb):
                d.wait()
        idx_copy(j + 1, lax.rem(j + 1, 4)).wait()
        for d in g_copies(j + 1, nb):
            d.start()
        for d in g_copies(j, b):
            d.wait()
        wb = pl.multiple_of(b * WIN, WIN)
        for g in range(WIN // 16):
            sl = pl.ds(g * 16, 16)
            al = asv[b, sl] + adv[b, sl]
            al = jnp.where(al >= 0, al, 0.2 * al)
            wwin[pl.ds(wb + g * 16, 16)] = jnp.exp(al - gvec)
        def e_body(e, carry2):
            we = plsc.load_gather(wwin, [jnp.full((16,), wb + e,
                                                  jnp.int32)])
            for g2 in range(D // 16):
                sl2 = pl.ds(g2 * 16, 16)
                rbuf[b, e, sl2] = rbuf[b, e, sl2] * we
            return carry2
        lax.fori_loop(0, WIN, e_body, 0, unroll=False)
        for d in s_copies(j, b):
            d.start()
        idx_copy(j + 3, lax.rem(j + 3, 4)).start()
        return carry

    lax.fori_loop(0, NWIN, win_body, 0, unroll=False)
    for d in s_copies(NWIN - 1, lax.rem(NWIN - 1, 2)):
        d.wait()
    for d in g_copies(NWIN, lax.rem(NWIN, 2)):
        d.wait()
    idx_copy(NWIN + 1, lax.rem(NWIN + 1, 4)).wait()
    idx_copy(NWIN + 2, lax.rem(NWIN + 2, 4)).wait()
    plsc.subcore_barrier()
    pltpu.sync_copy(acc_sh.at[pl.ds(s * RPS, RPS)],
                    acc_out.at[c].at[pl.ds(s * RPS, RPS)])
    pltpu.sync_copy(den_sh.at[pl.ds(s * RPS, RPS)],
                    den_out.at[c].at[pl.ds(s * RPS, RPS)])


_sc_layer = pl.kernel(
    _sc_body,
    out_type=(
        jax.ShapeDtypeStruct((NC, NROW, D), jnp.float32),
        jax.ShapeDtypeStruct((NC, NROW), jnp.float32),
    ),
    mesh=plsc.VectorSubcoreMesh(core_axis_name="c", subcore_axis_name="s",
                                num_cores=NC, num_subcores=NS),
    compiler_params=pltpu.CompilerParams(needs_layout_passes=False),
    scratch_types=[
        pltpu.VMEM((4, 2, WIN), jnp.int32),
        pltpu.VMEM((2, WIN, D), jnp.float32),
        pltpu.VMEM((2, WIN), jnp.float32),
        pltpu.VMEM((2, WIN), jnp.float32),
        pltpu.VMEM((16,), jnp.float32),
        pltpu.VMEM((2 * WIN,), jnp.float32),
        pltpu.VMEM_SHARED((NROW, D), jnp.float32),
        pltpu.VMEM_SHARED((NROW,), jnp.float32),
        pltpu.SemaphoreType.DMA((4,)),
        pltpu.SemaphoreType.DMA((2,)),
        pltpu.SemaphoreType.DMA((2,)),
    ],
)


def kernel(x, edge_index, edge_attr, batch,
           W1, b1, as1, ad1, W2, b2, as2, ad2, W3, b3, as3, ad3,
           lin_W, lin_b):
    del edge_attr
    src = edge_index[0].astype(jnp.int32).reshape(NW, EPW)
    dst = edge_index[1].astype(jnp.int32).reshape(NW, EPW)
    npad = NWX * WIN - EPW
    pad_src = jnp.zeros((NW, npad), jnp.int32)
    pad_dst = jnp.broadcast_to(
        N_NODES + (jnp.arange(npad, dtype=jnp.int32) % (NSC - N_NODES)),
        (NW, npad))
    srcw = jnp.concatenate([src, pad_src], axis=1).reshape(NW, NWX, WIN)
    dstw = jnp.concatenate([dst, pad_dst], axis=1).reshape(NW, NWX, WIN)
    eidx = jnp.stack([srcw, dstw], axis=2)
    zacc = jnp.zeros((NROW, D), jnp.float32)
    zden = jnp.zeros((NROW,), jnp.float32)
    batch_i32 = batch.astype(jnp.int32)

    h, asrc, adst, gub = _tc_first(x, W1, as1, ad1)
    acc, den = _sc_layer(h, asrc, adst, gub, eidx, zacc, zden)
    h, asrc, adst, gub = _tc_mid(acc, den, b1, W2, as2, ad2)
    acc, den = _sc_layer(h, asrc, adst, gub, eidx, zacc, zden)
    h, asrc, adst, gub = _tc_mid(acc, den, b2, W3, as3, ad3)
    acc, den = _sc_layer(h, asrc, adst, gub, eidx, zacc, zden)
    return _tc_final(acc, den, b3, batch_i32, lin_W, lin_b)


# speedup vs baseline: 1.3863x; 1.2929x over previous
import functools

import jax
import jax.numpy as jnp
from jax import lax
from jax.experimental import pallas as pl
from jax.experimental.pallas import tpu as pltpu
from jax.experimental.pallas import tpu_sc as plsc

N_NODES = 10000
N_EDGES = 320000
D = 128
N_GRAPHS = 64

NC = 2
NS = 16
NW = NC * NS
EPW = N_EDGES // NW
WIN = 128
NWIN = -(-EPW // WIN)
EPW_PAD = NWIN * WIN
PAD = EPW_PAD - EPW
NSC = N_NODES + 16
NROW = 10112
RPS = NROW // NS
DE = 144
WCOL = 128


def _pack_h(h, asrc):
    zpad = jnp.zeros((N_NODES, DE - D - 1), jnp.float32)
    return jnp.concatenate([h, asrc[:, None], zpad], axis=1)


def _tc_first_body(x_ref, w_ref, as_ref, ad_ref, h_ref, adst_ref, gub_ref):
    h = jnp.dot(x_ref[...], w_ref[...], preferred_element_type=jnp.float32)
    asrc = jnp.dot(h, as_ref[...], preferred_element_type=jnp.float32)
    adst = jnp.dot(h, ad_ref[...], preferred_element_type=jnp.float32)
    h_ref[...] = _pack_h(h, asrc)
    pad = jnp.zeros((NSC - N_NODES,), jnp.float32)
    adst_ref[...] = jnp.concatenate([adst, pad])
    ub = jnp.max(asrc) + jnp.max(adst)
    gub = jnp.where(ub >= 0, ub, 0.2 * ub)
    gub_ref[...] = jnp.full((128,), gub, jnp.float32)


def _tc_mid_body(acc_ref, b_ref, w_ref, as_ref, ad_ref,
                 h_ref, adst_ref, gub_ref):
    accs = acc_ref[0, :N_NODES, :D] + acc_ref[1, :N_NODES, :D]
    dens = acc_ref[0, :N_NODES, WCOL] + acc_ref[1, :N_NODES, WCOL]
    prev = accs / (dens + 1e-16)[:, None] + b_ref[...][None, :]
    prev = jnp.maximum(prev, 0.0)
    h = jnp.dot(prev, w_ref[...], preferred_element_type=jnp.float32)
    asrc = jnp.dot(h, as_ref[...], preferred_element_type=jnp.float32)
    adst = jnp.dot(h, ad_ref[...], preferred_element_type=jnp.float32)
    h_ref[...] = _pack_h(h, asrc)
    pad = jnp.zeros((NSC - N_NODES,), jnp.float32)
    adst_ref[...] = jnp.concatenate([adst, pad])
    ub = jnp.max(asrc) + jnp.max(adst)
    gub = jnp.where(ub >= 0, ub, 0.2 * ub)
    gub_ref[...] = jnp.full((128,), gub, jnp.float32)


def _tc_final_body(acc_ref, b_ref, batch_ref, lw_ref, lb_ref,
                   out_ref):
    accs = acc_ref[0, :N_NODES, :D] + acc_ref[1, :N_NODES, :D]
    dens = acc_ref[0, :N_NODES, WCOL] + acc_ref[1, :N_NODES, WCOL]
    node = accs / (dens + 1e-16)[:, None]
    gids = lax.broadcasted_iota(jnp.int32, (N_NODES, N_GRAPHS), 1)
    onehot = (batch_ref[...][:, None] == gids).astype(jnp.float32)
    pooled = lax.dot_general(onehot, node, (((0,), (0,)), ((), ())),
                             preferred_element_type=jnp.float32)
    cnt = jnp.sum(onehot, axis=0)
    pooled = pooled / jnp.maximum(cnt, 1.0)[:, None] + b_ref[...][None, :]
    out_ref[...] = (jnp.dot(pooled, lw_ref[...],
                            preferred_element_type=jnp.float32)
                    + lb_ref[...][None, :])


_TC_PARAMS = pltpu.CompilerParams(vmem_limit_bytes=100 * 1024 * 1024)


def _tc_first(x, w, a_s, a_d):
    return pl.pallas_call(
        _tc_first_body,
        out_shape=(
            jax.ShapeDtypeStruct((N_NODES, DE), jnp.float32),
            jax.ShapeDtypeStruct((NSC,), jnp.float32),
            jax.ShapeDtypeStruct((128,), jnp.float32),
        ),
        compiler_params=_TC_PARAMS,
    )(x, w, a_s, a_d)


def _tc_mid(acc, b, w, a_s, a_d):
    return pl.pallas_call(
        _tc_mid_body,
        out_shape=(
            jax.ShapeDtypeStruct((N_NODES, DE), jnp.float32),
            jax.ShapeDtypeStruct((NSC,), jnp.float32),
            jax.ShapeDtypeStruct((128,), jnp.float32),
        ),
        compiler_params=_TC_PARAMS,
    )(acc, b, w, a_s, a_d)


def _tc_final(acc, b, batch_i32, lin_w, lin_b):
    return pl.pallas_call(
        _tc_final_body,
        out_shape=jax.ShapeDtypeStruct((N_GRAPHS, D), jnp.float32),
        compiler_params=_TC_PARAMS,
    )(acc, b, batch_i32, lin_w, lin_b)


def _sc_body(h_hbm, adst_hbm, gub_hbm, srcw_hbm, dstw_hbm,
             zacc_hbm, acc_out,
             sidx, didx, adv, gub_v, rows, acc_sh, sem):
    c = lax.axis_index("c")
    s = lax.axis_index("s")
    w = c * NS + s
    pltpu.sync_copy(gub_hbm.at[pl.ds(0, 16)], gub_v)
    pltpu.sync_copy(srcw_hbm.at[w], sidx)
    pltpu.sync_copy(dstw_hbm.at[w], didx)
    pltpu.sync_copy(zacc_hbm.at[pl.ds(s * RPS, RPS)],
                    acc_sh.at[pl.ds(s * RPS, RPS)])
    plsc.subcore_barrier()
    gvec = gub_v[...]
    col_w = jnp.full((16,), WCOL, jnp.int32)

    def win_body(j, carry):
        d1 = pltpu.async_copy(h_hbm.at[sidx.at[j]], rows, sem)
        d2 = pltpu.async_copy(adst_hbm.at[didx.at[j]], adv, sem)
        d1.wait()
        d2.wait()
        for g in range(WIN // 16):
            e16 = lax.iota(jnp.int32, 16) + (g * 16)
            al = plsc.load_gather(rows, [e16, col_w]) + adv[pl.ds(g * 16, 16)]
            al = jnp.where(al >= 0, al, 0.2 * al)
            plsc.store_scatter(rows, [e16, col_w], jnp.exp(al - gvec))
        def e_body(e, carry2):
            we = plsc.load_gather(rows, [jnp.full((16,), e, jnp.int32),
                                         col_w])
            for g2 in range(D // 16):
                sl2 = pl.ds(g2 * 16, 16)
                rows[e, sl2] = rows[e, sl2] * we
            return carry2
        lax.fori_loop(0, WIN, e_body, 0, unroll=False)
        pltpu.sync_copy(rows, acc_sh.at[didx.at[j]], add=True)
        return carry

    lax.fori_loop(0, NWIN, win_body, 0, unroll=False)
    plsc.subcore_barrier()
    pltpu.sync_copy(acc_sh.at[pl.ds(s * RPS, RPS)],
                    acc_out.at[c].at[pl.ds(s * RPS, RPS)])


_sc_layer = pl.kernel(
    _sc_body,
    out_type=jax.ShapeDtypeStruct((NC, NROW, DE), jnp.float32),
    mesh=plsc.VectorSubcoreMesh(core_axis_name="c", subcore_axis_name="s",
                                num_cores=NC, num_subcores=NS),
    compiler_params=pltpu.CompilerParams(needs_layout_passes=False,
                                         use_tc_tiling_on_sc=False),
    scratch_types=[
        pltpu.VMEM((NWIN, WIN), jnp.int32),
        pltpu.VMEM((NWIN, WIN), jnp.int32),
        pltpu.VMEM((WIN,), jnp.float32),
        pltpu.VMEM((16,), jnp.float32),
        pltpu.VMEM((WIN, DE), jnp.float32),
        pltpu.VMEM_SHARED((NROW, DE), jnp.float32),
        pltpu.SemaphoreType.DMA,
    ],
)


def kernel(x, edge_index, edge_attr, batch,
           W1, b1, as1, ad1, W2, b2, as2, ad2, W3, b3, as3, ad3,
           lin_W, lin_b):
    del edge_attr
    src = edge_index[0].astype(jnp.int32).reshape(NW, EPW)
    dst = edge_index[1].astype(jnp.int32).reshape(NW, EPW)
    pad_src = jnp.zeros((NW, PAD), jnp.int32)
    pad_dst = jnp.broadcast_to(
        N_NODES + (jnp.arange(PAD, dtype=jnp.int32) % (NSC - N_NODES)),
        (NW, PAD))
    srcw = jnp.concatenate([src, pad_src], axis=1).reshape(NW, NWIN, WIN)
    dstw = jnp.concatenate([dst, pad_dst], axis=1).reshape(NW, NWIN, WIN)
    zacc = jnp.zeros((NROW, DE), jnp.float32)
    batch_i32 = batch.astype(jnp.int32)

    h, adst, gub = _tc_first(x, W1, as1, ad1)
    acc = _sc_layer(h, adst, gub, srcw, dstw, zacc)
    h, adst, gub = _tc_mid(acc, b1, W2, as2, ad2)
    acc = _sc_layer(h, adst, gub, srcw, dstw, zacc)
    h, adst, gub = _tc_mid(acc, b2, W3, as3, ad3)
    acc = _sc_layer(h, adst, gub, srcw, dstw, zacc)
    return _tc_final(acc, b3, batch_i32, lin_W, lin_b)


# speedup vs baseline: 1.5749x; 1.1361x over previous
import functools

import jax
import jax.numpy as jnp
from jax import lax
from jax.experimental import pallas as pl
from jax.experimental.pallas import tpu as pltpu
from jax.experimental.pallas import tpu_sc as plsc

N_NODES = 10000
N_EDGES = 320000
D = 128
N_GRAPHS = 64

NC = 2
NS = 16
NW = NC * NS
EPW = N_EDGES // NW
WIN = 128
NWIN = -(-EPW // WIN)
EPW_PAD = NWIN * WIN
PAD = EPW_PAD - EPW
NSC = N_NODES + 16
NROW = 10240
RPS = NROW // NS


def _tc_first_body(x_ref, w_ref, as_ref, ad_ref, h_ref, asrc_ref, adst_ref,
                   gub_ref):
    h = jnp.dot(x_ref[...], w_ref[...], preferred_element_type=jnp.float32)
    h_ref[...] = h
    asrc = jnp.dot(h, as_ref[...], preferred_element_type=jnp.float32)
    adst = jnp.dot(h, ad_ref[...], preferred_element_type=jnp.float32)
    pad = jnp.zeros((NSC - N_NODES,), jnp.float32)
    asrc_ref[...] = jnp.concatenate([asrc, pad])
    adst_ref[...] = jnp.concatenate([adst, pad])
    ub = jnp.max(asrc) + jnp.max(adst)
    gub = jnp.where(ub >= 0, ub, 0.2 * ub)
    gub_ref[...] = jnp.full((128,), gub, jnp.float32)


def _tc_mid_body(acc_ref, den_ref, b_ref, w_ref, as_ref, ad_ref,
                 h_ref, asrc_ref, adst_ref, gub_ref):
    accs = acc_ref[0, :N_NODES, :] + acc_ref[1, :N_NODES, :]
    dens = den_ref[0, :N_NODES] + den_ref[1, :N_NODES]
    prev = accs / (dens + 1e-16)[:, None] + b_ref[...][None, :]
    prev = jnp.maximum(prev, 0.0)
    h = jnp.dot(prev, w_ref[...], preferred_element_type=jnp.float32)
    h_ref[...] = h
    asrc = jnp.dot(h, as_ref[...], preferred_element_type=jnp.float32)
    adst = jnp.dot(h, ad_ref[...], preferred_element_type=jnp.float32)
    pad = jnp.zeros((NSC - N_NODES,), jnp.float32)
    asrc_ref[...] = jnp.concatenate([asrc, pad])
    adst_ref[...] = jnp.concatenate([adst, pad])
    ub = jnp.max(asrc) + jnp.max(adst)
    gub = jnp.where(ub >= 0, ub, 0.2 * ub)
    gub_ref[...] = jnp.full((128,), gub, jnp.float32)


def _tc_final_body(acc_ref, den_ref, b_ref, batch_ref, lw_ref, lb_ref,
                   out_ref):
    accs = acc_ref[0, :N_NODES, :] + acc_ref[1, :N_NODES, :]
    dens = den_ref[0, :N_NODES] + den_ref[1, :N_NODES]
    node = accs / (dens + 1e-16)[:, None]
    gids = lax.broadcasted_iota(jnp.int32, (N_NODES, N_GRAPHS), 1)
    onehot = (batch_ref[...][:, None] == gids).astype(jnp.float32)
    pooled = lax.dot_general(onehot, node, (((0,), (0,)), ((), ())),
                             preferred_element_type=jnp.float32)
    cnt = jnp.sum(onehot, axis=0)
    pooled = pooled / jnp.maximum(cnt, 1.0)[:, None] + b_ref[...][None, :]
    out_ref[...] = (jnp.dot(pooled, lw_ref[...],
                            preferred_element_type=jnp.float32)
                    + lb_ref[...][None, :])


_TC_PARAMS = pltpu.CompilerParams(vmem_limit_bytes=100 * 1024 * 1024)


def _tc_first(x, w, a_s, a_d):
    return pl.pallas_call(
        _tc_first_body,
        out_shape=(
            jax.ShapeDtypeStruct((N_NODES, D), jnp.float32),
            jax.ShapeDtypeStruct((NSC,), jnp.float32),
            jax.ShapeDtypeStruct((NSC,), jnp.float32),
            jax.ShapeDtypeStruct((128,), jnp.float32),
        ),
        compiler_params=_TC_PARAMS,
    )(x, w, a_s, a_d)


def _tc_mid(acc, den, b, w, a_s, a_d):
    return pl.pallas_call(
        _tc_mid_body,
        out_shape=(
            jax.ShapeDtypeStruct((N_NODES, D), jnp.float32),
            jax.ShapeDtypeStruct((NSC,), jnp.float32),
            jax.ShapeDtypeStruct((NSC,), jnp.float32),
            jax.ShapeDtypeStruct((128,), jnp.float32),
        ),
        compiler_params=_TC_PARAMS,
    )(acc, den, b, w, a_s, a_d)


def _tc_final(acc, den, b, batch_i32, lin_w, lin_b):
    return pl.pallas_call(
        _tc_final_body,
        out_shape=jax.ShapeDtypeStruct((N_GRAPHS, D), jnp.float32),
        compiler_params=_TC_PARAMS,
    )(acc, den, b, batch_i32, lin_w, lin_b)


def _sc_body(h_hbm, asrc_hbm, adst_hbm, gub_hbm, srcw_hbm, dstw_hbm,
             zacc_hbm, zden_hbm, acc_out, den_out,
             sidx, didx, asv, adv, gub_v, wwin, rows, acc_sh, den_sh,
             sem, rsem):
    c = lax.axis_index("c")
    s = lax.axis_index("s")
    w = c * NS + s
    pltpu.sync_copy(gub_hbm.at[pl.ds(0, 16)], gub_v)
    pltpu.sync_copy(srcw_hbm.at[w], sidx)
    pltpu.sync_copy(dstw_hbm.at[w], didx)
    pltpu.sync_copy(zacc_hbm.at[pl.ds(s * RPS, RPS)],
                    acc_sh.at[pl.ds(s * RPS, RPS)])
    pltpu.sync_copy(zden_hbm.at[pl.ds(s * RPS, RPS)],
                    den_sh.at[pl.ds(s * RPS, RPS)])
    plsc.subcore_barrier()
    gvec = gub_v[...]

    def win_body(j, carry):
        d2 = pltpu.async_copy(asrc_hbm.at[sidx.at[j]], asv, sem)
        d3 = pltpu.async_copy(adst_hbm.at[didx.at[j]], adv, sem)
        d1 = pltpu.async_copy(h_hbm.at[sidx.at[j]], rows, rsem)
        d2.wait()
        d3.wait()
        for g in range(WIN // 16):
            sl = pl.ds(g * 16, 16)
            al = asv[sl] + adv[sl]
            al = jnp.where(al >= 0, al, 0.2 * al)
            wwin[sl] = jnp.exp(al - gvec)
        d1.wait()
        def e_body(e, carry2):
            we = plsc.load_gather(wwin, [jnp.full((16,), e, jnp.int32)])
            for g2 in range(D // 16):
                sl2 = pl.ds(g2 * 16, 16)
                rows[e, sl2] = rows[e, sl2] * we
            return carry2
        lax.fori_loop(0, WIN, e_body, 0, unroll=False)
        pltpu.sync_copy(rows, acc_sh.at[didx.at[j]], add=True)
        pltpu.sync_copy(wwin, den_sh.at[didx.at[j]], add=True)
        return carry

    lax.fori_loop(0, NWIN, win_body, 0, unroll=False)
    plsc.subcore_barrier()
    pltpu.sync_copy(acc_sh.at[pl.ds(s * RPS, RPS)],
                    acc_out.at[c].at[pl.ds(s * RPS, RPS)])
    pltpu.sync_copy(den_sh.at[pl.ds(s * RPS, RPS)],
                    den_out.at[c].at[pl.ds(s * RPS, RPS)])


_sc_layer = pl.kernel(
    _sc_body,
    out_type=(
        jax.ShapeDtypeStruct((NC, NROW, D), jnp.float32),
        jax.ShapeDtypeStruct((NC, NROW), jnp.float32),
    ),
    mesh=plsc.VectorSubcoreMesh(core_axis_name="c", subcore_axis_name="s",
                                num_cores=NC, num_subcores=NS),
    compiler_params=pltpu.CompilerParams(needs_layout_passes=False),
    scratch_types=[
        pltpu.VMEM((NWIN, WIN), jnp.int32),
        pltpu.VMEM((NWIN, WIN), jnp.int32),
        pltpu.VMEM((WIN,), jnp.float32),
        pltpu.VMEM((WIN,), jnp.float32),
        pltpu.VMEM((16,), jnp.float32),
        pltpu.VMEM((WIN,), jnp.float32),
        pltpu.VMEM((WIN, D), jnp.float32),
        pltpu.VMEM_SHARED((NROW, D), jnp.float32),
        pltpu.VMEM_SHARED((NROW,), jnp.float32),
        pltpu.SemaphoreType.DMA,
        pltpu.SemaphoreType.DMA,
    ],
)


def kernel(x, edge_index, edge_attr, batch,
           W1, b1, as1, ad1, W2, b2, as2, ad2, W3, b3, as3, ad3,
           lin_W, lin_b):
    del edge_attr
    src = edge_index[0].astype(jnp.int32).reshape(NW, EPW)
    dst = edge_index[1].astype(jnp.int32).reshape(NW, EPW)
    pad_src = jnp.zeros((NW, PAD), jnp.int32)
    pad_dst = jnp.broadcast_to(
        N_NODES + (jnp.arange(PAD, dtype=jnp.int32) % (NSC - N_NODES)),
        (NW, PAD))
    srcw = jnp.concatenate([src, pad_src], axis=1).reshape(NW, NWIN, WIN)
    dstw = jnp.concatenate([dst, pad_dst], axis=1).reshape(NW, NWIN, WIN)
    zacc = jnp.zeros((NROW, D), jnp.float32)
    zden = jnp.zeros((NROW,), jnp.float32)
    batch_i32 = batch.astype(jnp.int32)

    h, asrc, adst, gub = _tc_first(x, W1, as1, ad1)
    acc, den = _sc_layer(h, asrc, adst, gub, srcw, dstw, zacc, zden)
    h, asrc, adst, gub = _tc_mid(acc, den, b1, W2, as2, ad2)
    acc, den = _sc_layer(h, asrc, adst, gub, srcw, dstw, zacc, zden)
    h, asrc, adst, gub = _tc_mid(acc, den, b2, W3, as3, ad3)
    acc, den = _sc_layer(h, asrc, adst, gub, srcw, dstw, zacc, zden)
    return _tc_final(acc, den, b3, batch_i32, lin_W, lin_b)
